# baseline (speedup 1.0000x reference)
import functools

import jax
import jax.numpy as jnp
from jax import lax
from jax.experimental import pallas as pl
from jax.experimental.pallas import tpu as pltpu
from jax.experimental.pallas import tpu_sc as plsc

_N = 10000
_E = 320000
_D = 128
_DOUT = 40
_LANES = 128
_NSC = 2
_NSUB = 16
_NTILES = _NSC * _NSUB
_RPT = (-(-_E // (_LANES * _NTILES)) + 7) // 8 * 8
_ROWS = _RPT * _NTILES
_EPAD = _ROWS * _LANES
_ACC_ROWS = 10240
_RPT_ACC = _ACC_ROWS // _NSUB
_DEGW = 16
_ICH = 8
_R0 = 120
_R1 = _RPT * 2 - _R0
_NS0 = _R0 // _ICH
_NS1 = _R1 // _ICH

_f32 = jnp.float32


def _sc_mesh():
    return plsc.VectorSubcoreMesh(core_axis_name="c", subcore_axis_name="s")


def _sc_degrees(dstp):
    @functools.partial(
        pl.kernel,
        mesh=_sc_mesh(),
        out_type=jax.ShapeDtypeStruct((_NSC, _ACC_ROWS, _DEGW), _f32),
        scratch_types=[
            pltpu.VMEM((_RPT, _LANES), jnp.int32),
            pltpu.VMEM((_LANES, _DEGW), _f32),
            pltpu.VMEM_SHARED((_ACC_ROWS, _DEGW), _f32),
        ],
    )
    def kern(dstp_hbm, out_hbm, dst_v, buf_v, acc_sh):
        c = lax.axis_index("c")
        s = lax.axis_index("s")
        w = c * _NSUB + s

        def fill(val):
            def row(i, carry):
                buf_v[i, pl.ds(0, _DEGW)] = jnp.full((_DEGW,), val, _f32)
                return carry
            lax.fori_loop(0, _LANES, row, 0)

        fill(0.0)

        def zacc(k, carry):
            pltpu.sync_copy(buf_v, acc_sh.at[pl.ds(s * _RPT_ACC + k * _LANES, _LANES)])
            return carry
        lax.fori_loop(0, _RPT_ACC // _LANES, zacc, 0)
        plsc.subcore_barrier()

        pltpu.sync_copy(dstp_hbm.at[pl.ds(w * _RPT, _RPT)], dst_v)
        fill(1.0)

        def body(j, carry):
            pltpu.sync_copy(buf_v, acc_sh.at[dst_v.at[j]], add=True)
            return carry
        lax.fori_loop(0, _RPT, body, 0)
        plsc.subcore_barrier()

        def wb(k, carry):
            r = s * _RPT_ACC + k * _LANES
            pltpu.sync_copy(acc_sh.at[pl.ds(r, _LANES)], buf_v)
            pltpu.sync_copy(buf_v, out_hbm.at[c, pl.ds(r, _LANES)])
            return carry
        lax.fori_loop(0, _RPT_ACC // _LANES, wb, 0)

    return kern(dstp)


def _sc_edge_pass(srcp, dstp, hs):
    @functools.partial(
        pl.kernel,
        mesh=_sc_mesh(),
        out_type=jax.ShapeDtypeStruct((_NSC, _ACC_ROWS, _D), _f32),
        scratch_types=[
            pltpu.VMEM((_ICH, _LANES), jnp.int32),
            pltpu.VMEM((_ICH, _LANES), jnp.int32),
            pltpu.VMEM((_ICH, _LANES), jnp.int32),
            pltpu.VMEM((_ICH, _LANES), jnp.int32),
            pltpu.VMEM((_LANES, _D), _f32),
            pltpu.VMEM((_LANES, _D), _f32),
            pltpu.VMEM_SHARED((_ACC_ROWS, _D), _f32),
            pltpu.SemaphoreType.DMA,
            pltpu.SemaphoreType.DMA,
            pltpu.SemaphoreType.DMA,
            pltpu.SemaphoreType.DMA,
        ],
    )
    def kern(srcp_hbm, dstp_hbm, hs_hbm, out_hbm,
             si0, si1, di0, di1, r0, r1, acc_sh, g0, g1, ss0, ss1):
        c = lax.axis_index("c")
        s = lax.axis_index("s")
        rows = (r0, r1)
        gsem = (g0, g1)
        ssem = (ss0, ss1)
        si = (si0, si1)
        di = (di0, di1)
        base = jnp.where(c == 0, s * _R0, _NSUB * _R0 + s * _R1)
        nsup = jnp.where(c == 0, _NS0, _NS1)
        nch = nsup * _ICH

        def load_idx(k_val, which):
            off = base + k_val * _ICH
            pltpu.sync_copy(srcp_hbm.at[pl.ds(off, _ICH)], si[which])
            pltpu.sync_copy(dstp_hbm.at[pl.ds(off, _ICH)], di[which])

        def gath(which, jj, b):
            return pltpu.make_async_copy(hs_hbm.at[si[which].at[jj]],
                                         rows[b], gsem[b])

        def scat(which, jj, b):
            return pltpu.make_async_copy(rows[b], acc_sh.at[di[which].at[jj]],
                                         ssem[b])

        with jax.named_scope("acc_zero"):
            def zrow(i, carry):
                def zcol(j, carry2):
                    r0[i, pl.ds(j * 16, 16)] = jnp.zeros((16,), _f32)
                    return carry2
                return lax.fori_loop(0, _D // 16, zcol, carry)
            lax.fori_loop(0, _LANES, zrow, 0)

            def zacc(k, carry):
                pltpu.sync_copy(r0, acc_sh.at[pl.ds(s * _RPT_ACC + k * _LANES, _LANES)])
                return carry
            lax.fori_loop(0, _RPT_ACC // _LANES, zacc, 0)
            plsc.subcore_barrier()

        load_idx(0, 0)
        gath(0, 0, 0).start()
        gath(0, 1, 1).start()

        def super_chunk(k_val, which):
            j0 = k_val * _ICH

            @pl.when(k_val + 1 < nsup)
            def _():
                load_idx(k_val + 1, 1 - which)
            for jj in range(0, _ICH, 2):
                for b in (0, 1):
                    gath(which, jj + b, b).wait()
                    pltpu.async_copy(rows[b], acc_sh.at[di[which].at[jj + b]],
                                     ssem[b], add=True)
                for b in (0, 1):
                    nxt = jj + 2 + b

                    @pl.when(j0 + nxt < nch)
                    def _():
                        scat(which, jj + b, b).wait()
                        if nxt < _ICH:
                            gath(which, nxt, b).start()
                        else:
                            gath(1 - which, nxt - _ICH, b).start()

        with jax.named_scope("edges"):
            def outer(k2, carry):
                super_chunk(2 * k2, 0)
                super_chunk(2 * k2 + 1, 1)
                return carry
            lax.fori_loop(0, (nsup - 1) // 2, outer, 0)
            super_chunk(nsup - 1, 0)
            for b in (0, 1):
                scat(0, b, b).wait()
            plsc.subcore_barrier()

        with jax.named_scope("writeback"):
            def wb(k, carry):
                r = s * _RPT_ACC + k * _LANES
                pltpu.sync_copy(acc_sh.at[pl.ds(r, _LANES)], r0)
                pltpu.sync_copy(r0, out_hbm.at[c, pl.ds(r, _LANES)])
                return carry
            lax.fori_loop(0, _RPT_ACC // _LANES, wb, 0)

    return kern(srcp, dstp, hs)


_BS = 1000


def _dense_in_body(x_ref, w_ref, dinv_ref, o_ref):
    h = jnp.dot(x_ref[...], w_ref[...], preferred_element_type=_f32)
    o_ref[...] = h * dinv_ref[...]


def _dense_mid_body(s0_ref, s1_ref, hs_ref, dinv_ref, b_ref, w_ref, o_ref):
    agg = s0_ref[...] + s1_ref[...] + hs_ref[...]
    z = jnp.maximum(b_ref[...] + dinv_ref[...] * agg, 0.0)
    o_ref[...] = jnp.dot(z, w_ref[...], preferred_element_type=_f32) * dinv_ref[...]


def _dense_out_body(s0_ref, s1_ref, hs_ref, dinv_ref, b_ref, wc_ref, bc_ref, o_ref):
    agg = s0_ref[...] + s1_ref[...] + hs_ref[...]
    z = jnp.maximum(b_ref[...] + dinv_ref[...] * agg, 0.0)
    o_ref[...] = jnp.dot(z, wc_ref[...], preferred_element_type=_f32) + bc_ref[...]


def _row_spec():
    return pl.BlockSpec((_BS, _D), lambda i: (i, 0))


def _full_spec():
    return pl.BlockSpec((_D, _D), lambda i: (0, 0))


def _dinv_spec():
    return pl.BlockSpec((_BS, 1), lambda i: (i, 0))


def _bias_spec():
    return pl.BlockSpec((1, _D), lambda i: (0, 0))


def _dense_in(x, w, dinv):
    return pl.pallas_call(
        _dense_in_body,
        grid=(_N // _BS,),
        in_specs=[_row_spec(), _full_spec(), _dinv_spec()],
        out_specs=_row_spec(),
        out_shape=jax.ShapeDtypeStruct((_N, _D), _f32),
    )(x, w, dinv)


def _dense_mid(s0, s1, hs, dinv, b, w):
    return pl.pallas_call(
        _dense_mid_body,
        grid=(_N // _BS,),
        in_specs=[_row_spec(), _row_spec(), _row_spec(), _dinv_spec(),
                  _bias_spec(), _full_spec()],
        out_specs=_row_spec(),
        out_shape=jax.ShapeDtypeStruct((_N, _D), _f32),
    )(s0, s1, hs, dinv, b, w)


def _dense_out(s0, s1, hs, dinv, b, wc, bc):
    return pl.pallas_call(
        _dense_out_body,
        grid=(_N // _BS,),
        in_specs=[_row_spec(), _row_spec(), _row_spec(), _dinv_spec(),
                  _bias_spec(), _full_spec(), _bias_spec()],
        out_specs=_row_spec(),
        out_shape=jax.ShapeDtypeStruct((_N, _D), _f32),
    )(s0, s1, hs, dinv, b, wc, bc)


def kernel(x, edge_index, W1, b1, W2, b2, W3, b3, Wc, bc):
    src = edge_index[0]
    dst = edge_index[1]
    pad = _EPAD - _E
    srcp = jnp.concatenate([src, jnp.zeros((pad,), jnp.int32)]).reshape(_ROWS, _LANES)
    dstp = jnp.concatenate([dst, jnp.full((pad,), _N, jnp.int32)]).reshape(_ROWS, _LANES)

    degp = _sc_degrees(dstp)
    deg = degp[0, :_N, 0] + degp[1, :_N, 0] + 1.0
    dinv = lax.rsqrt(deg)[:, None]

    hs1 = _dense_in(x, W1, dinv)
    agg1 = _sc_edge_pass(srcp, dstp, hs1)
    hs2 = _dense_mid(agg1[0, :_N], agg1[1, :_N], hs1, dinv, b1.reshape(1, _D), W2)
    agg2 = _sc_edge_pass(srcp, dstp, hs2)
    hs3 = _dense_mid(agg2[0, :_N], agg2[1, :_N], hs2, dinv, b2.reshape(1, _D), W3)
    agg3 = _sc_edge_pass(srcp, dstp, hs3)

    wcp = jnp.zeros((_D, _D), _f32).at[:, :_DOUT].set(Wc)
    bcp = jnp.zeros((1, _D), _f32).at[0, :_DOUT].set(bc)
    out = _dense_out(agg3[0, :_N], agg3[1, :_N], hs3, dinv, b3.reshape(1, _D), wcp, bcp)
    return out[:, :_DOUT]


# speedup vs baseline: 1.0285x; 1.0285x over previous
import functools

import jax
import jax.numpy as jnp
from jax import lax
from jax.experimental import pallas as pl
from jax.experimental.pallas import tpu as pltpu
from jax.experimental.pallas import tpu_sc as plsc

_N = 10000
_E = 320000
_D = 128
_DOUT = 40
_LANES = 128
_NSC = 2
_NSUB = 16
_NTILES = _NSC * _NSUB
_RPT = (-(-_E // (_LANES * _NTILES)) + 7) // 8 * 8
_ROWS = _RPT * _NTILES
_EPAD = _ROWS * _LANES
_ACC_ROWS = 10240
_RPT_ACC = _ACC_ROWS // _NSUB
_DEGW = 16
_ICH = 8
_R0 = 136
_R1 = _RPT * 2 - _R0
_NS0 = _R0 // _ICH
_NS1 = _R1 // _ICH

_f32 = jnp.float32


def _sc_mesh():
    return plsc.VectorSubcoreMesh(core_axis_name="c", subcore_axis_name="s")


def _sc_degrees(dstp):
    @functools.partial(
        pl.kernel,
        mesh=_sc_mesh(),
        out_type=jax.ShapeDtypeStruct((_NSC, _ACC_ROWS, _DEGW), _f32),
        scratch_types=[
            pltpu.VMEM((_RPT, _LANES), jnp.int32),
            pltpu.VMEM((_LANES, _DEGW), _f32),
            pltpu.VMEM_SHARED((_ACC_ROWS, _DEGW), _f32),
        ],
    )
    def kern(dstp_hbm, out_hbm, dst_v, buf_v, acc_sh):
        c = lax.axis_index("c")
        s = lax.axis_index("s")
        w = c * _NSUB + s

        def fill(val):
            def row(i, carry):
                buf_v[i, pl.ds(0, _DEGW)] = jnp.full((_DEGW,), val, _f32)
                return carry
            lax.fori_loop(0, _LANES, row, 0)

        fill(0.0)

        def zacc(k, carry):
            pltpu.sync_copy(buf_v, acc_sh.at[pl.ds(s * _RPT_ACC + k * _LANES, _LANES)])
            return carry
        lax.fori_loop(0, _RPT_ACC // _LANES, zacc, 0)
        plsc.subcore_barrier()

        pltpu.sync_copy(dstp_hbm.at[pl.ds(w * _RPT, _RPT)], dst_v)
        fill(1.0)

        def body(j, carry):
            pltpu.sync_copy(buf_v, acc_sh.at[dst_v.at[j]], add=True)
            return carry
        lax.fori_loop(0, _RPT, body, 0)
        plsc.subcore_barrier()

        def wb(k, carry):
            r = s * _RPT_ACC + k * _LANES
            pltpu.sync_copy(acc_sh.at[pl.ds(r, _LANES)], buf_v)
            pltpu.sync_copy(buf_v, out_hbm.at[c, pl.ds(r, _LANES)])
            return carry
        lax.fori_loop(0, _RPT_ACC // _LANES, wb, 0)

    return kern(dstp)


def _sc_edge_pass(srcp, dstp, hs):
    @functools.partial(
        pl.kernel,
        mesh=_sc_mesh(),
        out_type=jax.ShapeDtypeStruct((_NSC, _ACC_ROWS, _D), _f32),
        scratch_types=[
            pltpu.VMEM((_ICH, _LANES), jnp.int32),
            pltpu.VMEM((_ICH, _LANES), jnp.int32),
            pltpu.VMEM((_ICH, _LANES), jnp.int32),
            pltpu.VMEM((_ICH, _LANES), jnp.int32),
            pltpu.VMEM((_LANES, _D), _f32),
            pltpu.VMEM((_LANES, _D), _f32),
            pltpu.VMEM_SHARED((_ACC_ROWS, _D), _f32),
            pltpu.SemaphoreType.DMA,
            pltpu.SemaphoreType.DMA,
            pltpu.SemaphoreType.DMA,
            pltpu.SemaphoreType.DMA,
        ],
    )
    def kern(srcp_hbm, dstp_hbm, hs_hbm, out_hbm,
             si0, si1, di0, di1, r0, r1, acc_sh, g0, g1, ss0, ss1):
        c = lax.axis_index("c")
        s = lax.axis_index("s")
        rows = (r0, r1)
        gsem = (g0, g1)
        ssem = (ss0, ss1)
        si = (si0, si1)
        di = (di0, di1)
        base = jnp.where(c == 0, s * _R0, _NSUB * _R0 + s * _R1)
        nsup = jnp.where(c == 0, _NS0, _NS1)
        nch = nsup * _ICH

        def load_idx(k_val, which):
            off = base + k_val * _ICH
            pltpu.sync_copy(srcp_hbm.at[pl.ds(off, _ICH)], si[which])
            pltpu.sync_copy(dstp_hbm.at[pl.ds(off, _ICH)], di[which])

        def gath(which, jj, b):
            return pltpu.make_async_copy(hs_hbm.at[si[which].at[jj]],
                                         rows[b], gsem[b])

        def scat(which, jj, b):
            return pltpu.make_async_copy(rows[b], acc_sh.at[di[which].at[jj]],
                                         ssem[b])

        with jax.named_scope("acc_zero"):
            def zrow(i, carry):
                def zcol(j, carry2):
                    r0[i, pl.ds(j * 16, 16)] = jnp.zeros((16,), _f32)
                    return carry2
                return lax.fori_loop(0, _D // 16, zcol, carry)
            lax.fori_loop(0, _LANES, zrow, 0)

            def zacc(k, carry):
                pltpu.sync_copy(r0, acc_sh.at[pl.ds(s * _RPT_ACC + k * _LANES, _LANES)])
                return carry
            lax.fori_loop(0, _RPT_ACC // _LANES, zacc, 0)
            plsc.subcore_barrier()

        load_idx(0, 0)
        gath(0, 0, 0).start()
        gath(0, 1, 1).start()

        def super_chunk(k_val, which):
            j0 = k_val * _ICH

            @pl.when(k_val + 1 < nsup)
            def _():
                load_idx(k_val + 1, 1 - which)
            for jj in range(0, _ICH, 2):
                for b in (0, 1):
                    gath(which, jj + b, b).wait()
                    pltpu.async_copy(rows[b], acc_sh.at[di[which].at[jj + b]],
                                     ssem[b], add=True)
                for b in (0, 1):
                    nxt = jj + 2 + b

                    @pl.when(j0 + nxt < nch)
                    def _():
                        scat(which, jj + b, b).wait()
                        if nxt < _ICH:
                            gath(which, nxt, b).start()
                        else:
                            gath(1 - which, nxt - _ICH, b).start()

        with jax.named_scope("edges"):
            def outer(k2, carry):
                super_chunk(2 * k2, 0)
                super_chunk(2 * k2 + 1, 1)
                return carry
            lax.fori_loop(0, (nsup - 1) // 2, outer, 0)
            super_chunk(nsup - 1, 0)
            for b in (0, 1):
                scat(0, b, b).wait()
            plsc.subcore_barrier()

        with jax.named_scope("writeback"):
            def wb(k, carry):
                r = s * _RPT_ACC + k * _LANES
                pltpu.sync_copy(acc_sh.at[pl.ds(r, _LANES)], r0)
                pltpu.sync_copy(r0, out_hbm.at[c, pl.ds(r, _LANES)])
                return carry
            lax.fori_loop(0, _RPT_ACC // _LANES, wb, 0)

    return kern(srcp, dstp, hs)


_BS = 1000


def _dense_in_body(x_ref, w_ref, dinv_ref, o_ref):
    h = jnp.dot(x_ref[...], w_ref[...], preferred_element_type=_f32)
    o_ref[...] = h * dinv_ref[...]


def _dense_mid_body(s0_ref, s1_ref, hs_ref, dinv_ref, b_ref, w_ref, o_ref):
    agg = s0_ref[...] + s1_ref[...] + hs_ref[...]
    z = jnp.maximum(b_ref[...] + dinv_ref[...] * agg, 0.0)
    o_ref[...] = jnp.dot(z, w_ref[...], preferred_element_type=_f32) * dinv_ref[...]


def _dense_out_body(s0_ref, s1_ref, hs_ref, dinv_ref, b_ref, wc_ref, bc_ref, o_ref):
    agg = s0_ref[...] + s1_ref[...] + hs_ref[...]
    z = jnp.maximum(b_ref[...] + dinv_ref[...] * agg, 0.0)
    o_ref[...] = jnp.dot(z, wc_ref[...], preferred_element_type=_f32) + bc_ref[...]


def _row_spec():
    return pl.BlockSpec((_BS, _D), lambda i: (i, 0))


def _full_spec():
    return pl.BlockSpec((_D, _D), lambda i: (0, 0))


def _dinv_spec():
    return pl.BlockSpec((_BS, 1), lambda i: (i, 0))


def _bias_spec():
    return pl.BlockSpec((1, _D), lambda i: (0, 0))


def _dense_in(x, w, dinv):
    return pl.pallas_call(
        _dense_in_body,
        grid=(_N // _BS,),
        in_specs=[_row_spec(), _full_spec(), _dinv_spec()],
        out_specs=_row_spec(),
        out_shape=jax.ShapeDtypeStruct((_N, _D), _f32),
    )(x, w, dinv)


def _dense_mid(s0, s1, hs, dinv, b, w):
    return pl.pallas_call(
        _dense_mid_body,
        grid=(_N // _BS,),
        in_specs=[_row_spec(), _row_spec(), _row_spec(), _dinv_spec(),
                  _bias_spec(), _full_spec()],
        out_specs=_row_spec(),
        out_shape=jax.ShapeDtypeStruct((_N, _D), _f32),
    )(s0, s1, hs, dinv, b, w)


def _dense_out(s0, s1, hs, dinv, b, wc, bc):
    return pl.pallas_call(
        _dense_out_body,
        grid=(_N // _BS,),
        in_specs=[_row_spec(), _row_spec(), _row_spec(), _dinv_spec(),
                  _bias_spec(), _full_spec(), _bias_spec()],
        out_specs=_row_spec(),
        out_shape=jax.ShapeDtypeStruct((_N, _D), _f32),
    )(s0, s1, hs, dinv, b, wc, bc)


def kernel(x, edge_index, W1, b1, W2, b2, W3, b3, Wc, bc):
    src = edge_index[0]
    dst = edge_index[1]
    pad = _EPAD - _E
    srcp = jnp.concatenate([src, jnp.zeros((pad,), jnp.int32)]).reshape(_ROWS, _LANES)
    dstp = jnp.concatenate([dst, jnp.full((pad,), _N, jnp.int32)]).reshape(_ROWS, _LANES)

    degp = _sc_degrees(dstp)
    deg = degp[0, :_N, 0] + degp[1, :_N, 0] + 1.0
    dinv = lax.rsqrt(deg)[:, None]

    hs1 = _dense_in(x, W1, dinv)
    agg1 = _sc_edge_pass(srcp, dstp, hs1)
    hs2 = _dense_mid(agg1[0, :_N], agg1[1, :_N], hs1, dinv, b1.reshape(1, _D), W2)
    agg2 = _sc_edge_pass(srcp, dstp, hs2)
    hs3 = _dense_mid(agg2[0, :_N], agg2[1, :_N], hs2, dinv, b2.reshape(1, _D), W3)
    agg3 = _sc_edge_pass(srcp, dstp, hs3)

    wcp = jnp.zeros((_D, _D), _f32).at[:, :_DOUT].set(Wc)
    bcp = jnp.zeros((1, _D), _f32).at[0, :_DOUT].set(bc)
    out = _dense_out(agg3[0, :_N], agg3[1, :_N], hs3, dinv, b3.reshape(1, _D), wcp, bcp)
    return out[:, :_DOUT]


# speedup vs baseline: 1.6530x; 1.6072x over previous
import functools

import jax
import jax.numpy as jnp
from jax import lax
from jax.experimental import pallas as pl
from jax.experimental.pallas import tpu as pltpu
from jax.experimental.pallas import tpu_sc as plsc

_N = 10000
_E = 320000
_D = 128
_DOUT = 40
_LANES = 128
_NSC = 2
_NSUB = 16
_NTILES = _NSC * _NSUB
_RPT = (-(-_E // (_LANES * _NTILES)) + 7) // 8 * 8
_ROWS = _RPT * _NTILES
_EPAD = _ROWS * _LANES
_ACC_ROWS = 10240
_RPT_ACC = _ACC_ROWS // _NSUB
_DEGW = 16
_ICH = 8
_R0 = 136
_R1 = _RPT * 2 - _R0
_NS0 = _R0 // _ICH
_NS1 = _R1 // _ICH

_f32 = jnp.float32


def _sc_mesh():
    return plsc.VectorSubcoreMesh(core_axis_name="c", subcore_axis_name="s")


def _sc_degrees(dstp):
    @functools.partial(
        pl.kernel,
        mesh=_sc_mesh(),
        out_type=jax.ShapeDtypeStruct((_NSC, _ACC_ROWS, _DEGW), _f32),
        scratch_types=[
            pltpu.VMEM((_RPT, _LANES), jnp.int32),
            pltpu.VMEM((_LANES, _DEGW), _f32),
            pltpu.VMEM_SHARED((_ACC_ROWS, _DEGW), _f32),
        ],
    )
    def kern(dstp_hbm, out_hbm, dst_v, buf_v, acc_sh):
        c = lax.axis_index("c")
        s = lax.axis_index("s")
        w = c * _NSUB + s

        def fill(val):
            def row(i, carry):
                buf_v[i, pl.ds(0, _DEGW)] = jnp.full((_DEGW,), val, _f32)
                return carry
            lax.fori_loop(0, _LANES, row, 0)

        fill(0.0)

        def zacc(k, carry):
            pltpu.sync_copy(buf_v, acc_sh.at[pl.ds(s * _RPT_ACC + k * _LANES, _LANES)])
            return carry
        lax.fori_loop(0, _RPT_ACC // _LANES, zacc, 0)
        plsc.subcore_barrier()

        pltpu.sync_copy(dstp_hbm.at[pl.ds(w * _RPT, _RPT)], dst_v)
        fill(1.0)

        def body(j, carry):
            pltpu.sync_copy(buf_v, acc_sh.at[dst_v.at[j]], add=True)
            return carry
        lax.fori_loop(0, _RPT, body, 0)
        plsc.subcore_barrier()

        def wb(k, carry):
            r = s * _RPT_ACC + k * _LANES
            pltpu.sync_copy(acc_sh.at[pl.ds(r, _LANES)], buf_v)
            pltpu.sync_copy(buf_v, out_hbm.at[c, pl.ds(r, _LANES)])
            return carry
        lax.fori_loop(0, _RPT_ACC // _LANES, wb, 0)

    return kern(dstp)


def _sc_edge_pass(srcp, dstp, hs):
    @functools.partial(
        pl.kernel,
        mesh=_sc_mesh(),
        out_type=jax.ShapeDtypeStruct((_NSC, _ACC_ROWS, _D), _f32),
        scratch_types=[
            pltpu.VMEM((_ICH, _LANES), jnp.int32),
            pltpu.VMEM((_ICH, _LANES), jnp.int32),
            pltpu.VMEM((_ICH, _LANES), jnp.int32),
            pltpu.VMEM((_ICH, _LANES), jnp.int32),
            pltpu.VMEM((_LANES, _D), _f32),
            pltpu.VMEM((_LANES, _D), _f32),
            pltpu.VMEM_SHARED((_ACC_ROWS, _D), _f32),
            pltpu.SemaphoreType.DMA,
            pltpu.SemaphoreType.DMA,
            pltpu.SemaphoreType.DMA,
            pltpu.SemaphoreType.DMA,
        ],
    )
    def kern(srcp_hbm, dstp_hbm, hs_hbm, out_hbm,
             si0, si1, di0, di1, r0, r1, acc_sh, g0, g1, ss0, ss1):
        c = lax.axis_index("c")
        s = lax.axis_index("s")
        rows = (r0, r1)
        gsem = (g0, g1)
        ssem = (ss0, ss1)
        si = (si0, si1)
        di = (di0, di1)
        base = jnp.where(c == 0, s * _R0, _NSUB * _R0 + s * _R1)
        nsup = jnp.where(c == 0, _NS0, _NS1)
        nch = nsup * _ICH

        def load_idx(k_val, which):
            off = base + k_val * _ICH
            pltpu.sync_copy(srcp_hbm.at[pl.ds(off, _ICH)], si[which])
            pltpu.sync_copy(dstp_hbm.at[pl.ds(off, _ICH)], di[which])

        def gath(which, jj, b):
            return pltpu.make_async_copy(hs_hbm.at[si[which].at[jj]],
                                         rows[b], gsem[b])

        def scat(which, jj, b):
            return pltpu.make_async_copy(rows[b], acc_sh.at[di[which].at[jj]],
                                         ssem[b])

        with jax.named_scope("acc_zero"):
            def zrow(i, carry):
                def zcol(j, carry2):
                    r0[i, pl.ds(j * 16, 16)] = jnp.zeros((16,), _f32)
                    return carry2
                return lax.fori_loop(0, _D // 16, zcol, carry)
            lax.fori_loop(0, _LANES, zrow, 0)

            def zacc(k, carry):
                pltpu.sync_copy(r0, acc_sh.at[pl.ds(s * _RPT_ACC + k * _LANES, _LANES)])
                return carry
            lax.fori_loop(0, _RPT_ACC // _LANES, zacc, 0)
            plsc.subcore_barrier()

        load_idx(0, 0)
        gath(0, 0, 0).start()
        gath(0, 1, 1).start()

        def super_chunk(k_val, which):
            j0 = k_val * _ICH

            @pl.when(k_val + 1 < nsup)
            def _():
                load_idx(k_val + 1, 1 - which)
            for jj in range(0, _ICH, 2):
                for b in (0, 1):
                    gath(which, jj + b, b).wait()
                    pltpu.async_copy(rows[b], acc_sh.at[di[which].at[jj + b]],
                                     ssem[b], add=True)
                for b in (0, 1):
                    nxt = jj + 2 + b

                    @pl.when(j0 + nxt < nch)
                    def _():
                        scat(which, jj + b, b).wait()
                        if nxt < _ICH:
                            gath(which, nxt, b).start()
                        else:
                            gath(1 - which, nxt - _ICH, b).start()

        with jax.named_scope("edges"):
            def outer(k2, carry):
                super_chunk(2 * k2, 0)
                super_chunk(2 * k2 + 1, 1)
                return carry
            lax.fori_loop(0, (nsup - 1) // 2, outer, 0)
            super_chunk(nsup - 1, 0)
            for b in (0, 1):
                scat(0, b, b).wait()
            plsc.subcore_barrier()

        with jax.named_scope("writeback"):
            def wb(k, carry):
                r = s * _RPT_ACC + k * _LANES
                pltpu.sync_copy(acc_sh.at[pl.ds(r, _LANES)], r0)
                pltpu.sync_copy(r0, out_hbm.at[c, pl.ds(r, _LANES)])
                return carry
            lax.fori_loop(0, _RPT_ACC // _LANES, wb, 0)

    return kern(srcp, dstp, hs)


_BS = 1000


def _dense_in_body(x_ref, w_ref, dinv_ref, o_ref):
    h = jnp.dot(x_ref[...], w_ref[...], preferred_element_type=_f32)
    o_ref[...] = h * dinv_ref[...]


def _dense_mid_body(s0_ref, s1_ref, hs_ref, dinv_ref, b_ref, w_ref, o_ref):
    agg = s0_ref[...] + s1_ref[...] + hs_ref[...]
    z = jnp.maximum(b_ref[...] + dinv_ref[...] * agg, 0.0)
    o_ref[...] = jnp.dot(z, w_ref[...], preferred_element_type=_f32) * dinv_ref[...]


def _dense_out_body(s0_ref, s1_ref, hs_ref, dinv_ref, b_ref, wc_ref, bc_ref, o_ref):
    agg = s0_ref[...] + s1_ref[...] + hs_ref[...]
    z = jnp.maximum(b_ref[...] + dinv_ref[...] * agg, 0.0)
    o_ref[...] = jnp.dot(z, wc_ref[...], preferred_element_type=_f32) + bc_ref[...]


def _row_spec():
    return pl.BlockSpec((_BS, _D), lambda i: (i, 0))


def _full_spec():
    return pl.BlockSpec((_D, _D), lambda i: (0, 0))


def _dinv_spec():
    return pl.BlockSpec((_BS, 1), lambda i: (i, 0))


def _bias_spec():
    return pl.BlockSpec((1, _D), lambda i: (0, 0))


def _dense_in(x, w, dinv):
    return pl.pallas_call(
        _dense_in_body,
        grid=(_N // _BS,),
        in_specs=[_row_spec(), _full_spec(), _dinv_spec()],
        out_specs=_row_spec(),
        out_shape=jax.ShapeDtypeStruct((_N, _D), _f32),
    )(x, w, dinv)


def _dense_mid(s0, s1, hs, dinv, b, w):
    return pl.pallas_call(
        _dense_mid_body,
        grid=(_N // _BS,),
        in_specs=[_row_spec(), _row_spec(), _row_spec(), _dinv_spec(),
                  _bias_spec(), _full_spec()],
        out_specs=_row_spec(),
        out_shape=jax.ShapeDtypeStruct((_N, _D), _f32),
    )(s0, s1, hs, dinv, b, w)


def _dense_out(s0, s1, hs, dinv, b, wc, bc):
    return pl.pallas_call(
        _dense_out_body,
        grid=(_N // _BS,),
        in_specs=[_row_spec(), _row_spec(), _row_spec(), _dinv_spec(),
                  _bias_spec(), _full_spec(), _bias_spec()],
        out_specs=_row_spec(),
        out_shape=jax.ShapeDtypeStruct((_N, _D), _f32),
    )(s0, s1, hs, dinv, b, wc, bc)


def kernel(x, edge_index, W1, b1, W2, b2, W3, b3, Wc, bc):
    src = edge_index[0]
    dst = edge_index[1]
    pad = _EPAD - _E
    fill = jnp.arange(pad, dtype=jnp.int32)
    srcp = jnp.concatenate([src, fill % 128]).reshape(_ROWS, _LANES)
    dstp = jnp.concatenate([dst, _N + fill % (_ACC_ROWS - _N)]).reshape(_ROWS, _LANES)

    degp = _sc_degrees(dstp)
    deg = degp[0, :_N, 0] + degp[1, :_N, 0] + 1.0
    dinv = lax.rsqrt(deg)[:, None]

    hs1 = _dense_in(x, W1, dinv)
    agg1 = _sc_edge_pass(srcp, dstp, hs1)
    hs2 = _dense_mid(agg1[0, :_N], agg1[1, :_N], hs1, dinv, b1.reshape(1, _D), W2)
    agg2 = _sc_edge_pass(srcp, dstp, hs2)
    hs3 = _dense_mid(agg2[0, :_N], agg2[1, :_N], hs2, dinv, b2.reshape(1, _D), W3)
    agg3 = _sc_edge_pass(srcp, dstp, hs3)

    wcp = jnp.zeros((_D, _D), _f32).at[:, :_DOUT].set(Wc)
    bcp = jnp.zeros((1, _D), _f32).at[0, :_DOUT].set(bc)
    out = _dense_out(agg3[0, :_N], agg3[1, :_N], hs3, dinv, b3.reshape(1, _D), wcp, bcp)
    return out[:, :_DOUT]
